# baseline (speedup 1.0000x reference)
import jax
import jax.numpy as jnp
import numpy as np
from jax.experimental import pallas as pl

BT = 64


def _fused_kernel(x_ref, t1_ref, b1_ref, t2_ref, b2_ref,
                  fw1_ref, fb1_ref, fw2_ref, fb2_ref, out_ref):
    xb = x_ref[...].astype(jnp.bfloat16)

    acc1 = None
    for di in range(3):
        xd = xb[:, di:di + 26, :].reshape(BT * 26, 28)
        part = jnp.dot(xd, t1_ref[di], preferred_element_type=jnp.float32)
        acc1 = part if acc1 is None else acc1 + part
    h1 = acc1 + b1_ref[...]
    x1 = jnp.maximum(h1, 0.0).astype(jnp.bfloat16).reshape(BT, 26, 832)

    acc2 = None
    for di in range(3):
        xs = x1[:, di:di + 24, :].reshape(BT * 24, 832)
        part = jnp.dot(xs, t2_ref[di], preferred_element_type=jnp.float32)
        acc2 = part if acc2 is None else acc2 + part
    h2 = jnp.maximum(acc2 + b2_ref[...], 0.0)

    hr = jnp.max(h2.reshape(BT, 12, 2, 1536), axis=2)
    hrr = hr.reshape(BT, 12, 12, 128)
    p = jnp.maximum(hrr[..., :64], hrr[..., 64:])
    flat = p.reshape(BT, 9216).astype(jnp.bfloat16)

    f1 = jnp.dot(flat, fw1_ref[...], preferred_element_type=jnp.float32)
    f1 = jnp.maximum(f1 + fb1_ref[...], 0.0).astype(jnp.bfloat16)
    f2 = jnp.dot(f1, fw2_ref[...], preferred_element_type=jnp.float32)
    f2 = f2 + fb2_ref[...]
    out_ref[...] = f2[:, :10]


_E1 = np.zeros((3, 28, 26), np.float32)
_E2 = np.zeros((3, 26, 24), np.float32)
for _e in range(3):
    _E1[_e, np.arange(26) + _e, np.arange(26)] = 1.0
    _E2[_e, np.arange(24) + _e, np.arange(24)] = 1.0


def _prep_weights(conv1_w, conv1_b, conv2_w, conv2_b, fc1_w, fc1_b, fc2_w, fc2_b):
    w1r = conv1_w[:, 0, :, :]
    t1 = jnp.einsum('epj,cde->dpjc', _E1, w1r).reshape(3, 28, 832)
    t2 = jnp.einsum('ejq,oide->djiqo', _E2, conv2_w).reshape(3, 832, 1536)
    b1t = jnp.tile(conv1_b, 26).reshape(1, 832)
    b2t = jnp.tile(conv2_b, 24).reshape(1, 1536)
    fw1 = fc1_w.reshape(128, 64, 12, 12).transpose(0, 2, 3, 1)
    fw1 = fw1.reshape(128, 9216).T
    fw2 = jnp.zeros((128, 128), jnp.float32).at[:, :10].set(fc2_w.T)
    fb2 = jnp.zeros((1, 128), jnp.float32).at[0, :10].set(fc2_b)
    return (t1.astype(jnp.bfloat16), b1t, t2.astype(jnp.bfloat16), b2t,
            fw1.astype(jnp.bfloat16), fc1_b.reshape(1, 128),
            fw2.astype(jnp.bfloat16), fb2)


def _call(xr, args, interpret=False):
    b = xr.shape[0]
    grid = b // BT
    const = lambda *shape: pl.BlockSpec(shape, lambda i: (0,) * len(shape))
    return pl.pallas_call(
        _fused_kernel,
        grid=(grid,),
        in_specs=[
            pl.BlockSpec((BT, 28, 28), lambda i: (i, 0, 0)),
            const(3, 28, 832), const(1, 832), const(3, 832, 1536),
            const(1, 1536), const(9216, 128), const(1, 128),
            const(128, 128), const(1, 128),
        ],
        out_specs=pl.BlockSpec((BT, 10), lambda i: (i, 0)),
        out_shape=jax.ShapeDtypeStruct((b, 10), jnp.float32),
        interpret=interpret,
    )(xr, *args)


def kernel(x, conv1_w, conv1_b, conv2_w, conv2_b, fc1_w, fc1_b, fc2_w, fc2_b):
    args = _prep_weights(conv1_w, conv1_b, conv2_w, conv2_b,
                         fc1_w, fc1_b, fc2_w, fc2_b)
    xr = x.reshape(x.shape[0], 28, 28)
    return _call(xr, args)


# speedup vs baseline: 1.8331x; 1.8331x over previous
import functools
import jax
import jax.numpy as jnp
import numpy as np
from jax.experimental import pallas as pl
from jax.experimental.pallas import tpu as pltpu

BT = 256


def _net_kernel(xt_ref, t1_ref, b1_ref, t2_ref, b2_ref,
                fw1_ref, fb1_ref, fw2_ref, fb2_ref, out_ref,
                x1_scr, flat_scr):
    def c1_body(i, carry):
        acc = None
        for d in range(3):
            xi = xt_ref[pl.ds(i + d, 1)].reshape(28, BT).astype(jnp.bfloat16)
            part = jnp.dot(t1_ref[d], xi, preferred_element_type=jnp.float32)
            acc = part if acc is None else acc + part
        h = jnp.maximum(acc + b1_ref[...], 0.0).astype(jnp.bfloat16)
        x1_scr[pl.ds(i, 1)] = h.reshape(1, 832, BT)
        return carry

    jax.lax.fori_loop(0, 26, c1_body, 0, unroll=2)

    def c2_body(k, carry):
        rows = []
        for r in range(2):
            acc = None
            for d in range(3):
                xi = x1_scr[pl.ds(2 * k + r + d, 1)].reshape(832, BT)
                part = jnp.dot(t2_ref[d], xi,
                               preferred_element_type=jnp.float32)
                acc = part if acc is None else acc + part
            rows.append(jnp.maximum(acc + b2_ref[...], 0.0))
        m = jnp.maximum(rows[0], rows[1])
        m = m.reshape(12, 128, BT)
        m = jnp.maximum(m[:, :64, :], m[:, 64:, :])
        flat_scr[pl.ds(k * 768, 768)] = m.reshape(768, BT).astype(jnp.bfloat16)
        return carry

    jax.lax.fori_loop(0, 12, c2_body, 0, unroll=2)

    f1 = jnp.dot(fw1_ref[...], flat_scr[...],
                 preferred_element_type=jnp.float32)
    f1 = jnp.maximum(f1 + fb1_ref[...], 0.0).astype(jnp.bfloat16)
    f2 = jnp.dot(fw2_ref[...], f1, preferred_element_type=jnp.float32)
    out_ref[...] = f2 + fb2_ref[...]


_E1 = np.zeros((3, 28, 26), np.float32)
_E2 = np.zeros((3, 26, 24), np.float32)
for _e in range(3):
    _E1[_e, np.arange(26) + _e, np.arange(26)] = 1.0
    _E2[_e, np.arange(24) + _e, np.arange(24)] = 1.0


def _prep(conv1_w, conv1_b, conv2_w, conv2_b, fc1_w, fc1_b, fc2_w, fc2_b):
    w1r = conv1_w[:, 0, :, :]
    t1 = jnp.einsum('epj,cde->djcp', _E1, w1r).reshape(3, 832, 28)
    t2 = jnp.einsum('ejq,oide->dqoji', _E2, conv2_w).reshape(3, 1536, 832)
    b1 = jnp.broadcast_to(jnp.tile(conv1_b, 26)[:, None], (832, BT))
    b2 = jnp.broadcast_to(jnp.tile(conv2_b, 24)[:, None], (1536, BT))
    fw1 = fc1_w.reshape(128, 64, 12, 12).transpose(0, 2, 3, 1).reshape(128, 9216)
    fb1 = jnp.broadcast_to(fc1_b[:, None], (128, BT))
    fw2 = jnp.zeros((16, 128), jnp.float32).at[:10].set(fc2_w)
    fb2 = jnp.zeros((16,), jnp.float32).at[:10].set(fc2_b)
    fb2 = jnp.broadcast_to(fb2[:, None], (16, BT))
    return (t1.astype(jnp.bfloat16), b1, t2.astype(jnp.bfloat16), b2,
            fw1.astype(jnp.bfloat16), fb1, fw2.astype(jnp.bfloat16), fb2)


def _call(xt, args, interpret=False):
    b = xt.shape[2]
    grid = b // BT
    const = lambda *shape: pl.BlockSpec(shape, lambda i: (0,) * len(shape))
    return pl.pallas_call(
        _net_kernel,
        grid=(grid,),
        in_specs=[
            pl.BlockSpec((28, 28, BT), lambda i: (0, 0, i)),
            const(3, 832, 28), const(832, BT), const(3, 1536, 832),
            const(1536, BT), const(128, 9216), const(128, BT),
            const(16, 128), const(16, BT),
        ],
        out_specs=pl.BlockSpec((16, BT), lambda i: (0, i)),
        out_shape=jax.ShapeDtypeStruct((16, b), jnp.float32),
        scratch_shapes=[
            pltpu.VMEM((26, 832, BT), jnp.bfloat16),
            pltpu.VMEM((9216, BT), jnp.bfloat16),
        ],
        interpret=interpret,
    )(xt, *args)


def kernel(x, conv1_w, conv1_b, conv2_w, conv2_b, fc1_w, fc1_b, fc2_w, fc2_b):
    args = _prep(conv1_w, conv1_b, conv2_w, conv2_b,
                 fc1_w, fc1_b, fc2_w, fc2_b)
    xt = x.reshape(x.shape[0], 28, 28).transpose(1, 2, 0)
    out = _call(xt, args)
    return out[:10].T
